# Initial kernel scaffold; baseline (speedup 1.0000x reference)
import jax
import jax.numpy as jnp
from jax.experimental import pallas as pl


def kernel(lngs, lats, grid_id, emb_table, W, b):
    raise NotImplementedError("write your pallas kernel here")


# speedup vs baseline: 4.3726x; 4.3726x over previous
import functools

import jax
import jax.numpy as jnp
from jax import lax
from jax.experimental import pallas as pl
from jax.experimental.pallas import tpu as pltpu
from jax.experimental.pallas import tpu_sc as plsc

B, L = 4096, 200
VOCAB, EMB = 128 * 128, 32
N = B * L

NC, NS = 2, 16
NW = NC * NS
TOK_PER_W = N // NW
CHUNK = 1024
NCHUNK = TOK_PER_W // CHUNK
IDX_ROWS = CHUNK // 128


def _tc_fold_table(emb_table, W, b1):
    def body(emb_ref, w_ref, b_ref, out_ref):
        w2 = w_ref[2:2 + EMB, :] * 2.0
        acc = jnp.dot(emb_ref[...], w2, preferred_element_type=jnp.float32)
        out_ref[...] = acc + 2.0 * b_ref[...]

    return pl.pallas_call(
        body,
        out_shape=jax.ShapeDtypeStruct((VOCAB, EMB), jnp.float32),
    )(emb_table, W, b1)


def _sc_gather_affine_tanh(t2, ids2d, lng, lat, wc2):
    mesh = plsc.VectorSubcoreMesh(core_axis_name="c", subcore_axis_name="s")

    @functools.partial(
        pl.kernel,
        out_type=jax.ShapeDtypeStruct((N, EMB), jnp.float32),
        mesh=mesh,
        scratch_types=[
            pltpu.VMEM((IDX_ROWS, 128), jnp.int32),
            pltpu.VMEM((CHUNK, EMB), jnp.float32),
            pltpu.VMEM((CHUNK,), jnp.float32),
            pltpu.VMEM((CHUNK,), jnp.float32),
            pltpu.VMEM((2, EMB), jnp.float32),
            pltpu.SemaphoreType.DMA,
        ],
        compiler_params=pltpu.CompilerParams(use_tc_tiling_on_sc=False),
    )
    def k(t2_hbm, ids_hbm, lng_hbm, lat_hbm, wc_hbm, out_hbm,
          idx_v, rows_v, lng_v, lat_v, wc_v, sem):
        wid = lax.axis_index("s") * NC + lax.axis_index("c")
        base = wid * TOK_PER_W
        pltpu.sync_copy(wc_hbm, wc_v)
        w00 = wc_v[0, 0:16]
        w01 = wc_v[0, 16:32]
        w10 = wc_v[1, 0:16]
        w11 = wc_v[1, 16:32]

        def chunk_body(c, chunk_carry):
            bc = pl.multiple_of(base + c * CHUNK, CHUNK)
            bc_row = pl.multiple_of(base // 128 + c * IDX_ROWS, IDX_ROWS)
            pltpu.sync_copy(ids_hbm.at[pl.ds(bc_row, IDX_ROWS), :], idx_v)
            cps = [
                pltpu.async_copy(
                    t2_hbm.at[idx_v.at[j]],
                    rows_v.at[pl.ds(j * 128, 128)],
                    sem,
                )
                for j in range(IDX_ROWS)
            ]
            pltpu.sync_copy(lng_hbm.at[pl.ds(bc, CHUNK)], lng_v)
            pltpu.sync_copy(lat_hbm.at[pl.ds(bc, CHUNK)], lat_v)
            for cp in cps:
                cp.wait()

            def body(g, carry):
                tb = g * 16
                lng16 = lng_v[pl.ds(tb, 16)]
                lat16 = lat_v[pl.ds(tb, 16)]
                for j in range(16):
                    lng_s = lng16[j]
                    lat_s = lat16[j]
                    t = tb + j
                    g0 = rows_v[t, 0:16]
                    y0 = jnp.minimum(g0 + lng_s * w00 + lat_s * w10, 80.0)
                    e0 = jnp.exp(y0)
                    rows_v[t, 0:16] = (e0 - 1.0) / (e0 + 1.0)
                    g1 = rows_v[t, 16:32]
                    y1 = jnp.minimum(g1 + lng_s * w01 + lat_s * w11, 80.0)
                    e1 = jnp.exp(y1)
                    rows_v[t, 16:32] = (e1 - 1.0) / (e1 + 1.0)
                return carry

            lax.fori_loop(0, CHUNK // 16, body, 0)
            pltpu.sync_copy(rows_v, out_hbm.at[pl.ds(bc, CHUNK), :])
            return chunk_carry

        lax.fori_loop(0, NCHUNK, chunk_body, 0)

    return k(t2, ids2d, lng, lat, wc2)


def kernel(lngs, lats, grid_id, emb_table, W, b):
    t2 = _tc_fold_table(emb_table, W, b.reshape(1, EMB))
    wc2 = W[0:2, :] * 2.0
    ids2d = grid_id.astype(jnp.int32).reshape(N // 128, 128)
    out = _sc_gather_affine_tanh(
        t2, ids2d, lngs.reshape(N), lats.reshape(N), wc2)
    return out.reshape(B, L, EMB)


# speedup vs baseline: 4.4698x; 1.0222x over previous
import functools

import jax
import jax.numpy as jnp
from jax import lax
from jax.experimental import pallas as pl
from jax.experimental.pallas import tpu as pltpu
from jax.experimental.pallas import tpu_sc as plsc

B, L = 4096, 200
VOCAB, EMB = 128 * 128, 32
N = B * L

NC, NS = 2, 16
NW = NC * NS
TOK_PER_W = N // NW
CHUNK = 3200
CHUNK_ROWS = CHUNK // L
ROWS_PER_W = B // NW
NCHUNK = TOK_PER_W // CHUNK
IDX_ROWS = CHUNK // 128


def _tc_fold_table(emb_table, W, b1):
    def body(emb_ref, w_ref, b_ref, out_ref):
        w2 = w_ref[2:2 + EMB, :] * 2.0
        acc = jnp.dot(emb_ref[...], w2, preferred_element_type=jnp.float32)
        out_ref[...] = acc + 2.0 * b_ref[...]

    return pl.pallas_call(
        body,
        out_shape=jax.ShapeDtypeStruct((VOCAB, EMB), jnp.float32),
    )(emb_table, W, b1)


def _sc_gather_affine_tanh(t2, ids3d, lng, lat, wc2):
    mesh = plsc.VectorSubcoreMesh(core_axis_name="c", subcore_axis_name="s")

    @functools.partial(
        pl.kernel,
        out_type=jax.ShapeDtypeStruct((B, L, EMB), jnp.float32),
        mesh=mesh,
        scratch_types=[
            pltpu.VMEM((IDX_ROWS, 128), jnp.int32),
            pltpu.VMEM((CHUNK, EMB), jnp.float32),
            pltpu.VMEM((CHUNK,), jnp.float32),
            pltpu.VMEM((CHUNK,), jnp.float32),
            pltpu.VMEM((2, EMB), jnp.float32),
            pltpu.SemaphoreType.DMA,
        ],
        compiler_params=pltpu.CompilerParams(use_tc_tiling_on_sc=False),
    )
    def k(t2_hbm, ids_hbm, lng_hbm, lat_hbm, wc_hbm, out_hbm,
          idx_v, rows_v, lng_v, lat_v, wc_v, sem):
        wid = lax.axis_index("s") * NC + lax.axis_index("c")
        base = wid * TOK_PER_W
        row_base = wid * ROWS_PER_W
        pltpu.sync_copy(wc_hbm, wc_v)
        w00 = wc_v[0, 0:16]
        w01 = wc_v[0, 16:32]
        w10 = wc_v[1, 0:16]
        w11 = wc_v[1, 16:32]

        def chunk_body(c, chunk_carry):
            bc = pl.multiple_of(base + c * CHUNK, CHUNK)
            pltpu.sync_copy(ids_hbm.at[wid * NCHUNK + c], idx_v)
            cps = [
                pltpu.async_copy(
                    t2_hbm.at[idx_v.at[j]],
                    rows_v.at[pl.ds(j * 128, 128)],
                    sem,
                )
                for j in range(IDX_ROWS)
            ]
            pltpu.sync_copy(lng_hbm.at[pl.ds(bc, CHUNK)], lng_v)
            pltpu.sync_copy(lat_hbm.at[pl.ds(bc, CHUNK)], lat_v)
            for cp in cps:
                cp.wait()

            def body(g, carry):
                tb = g * 16
                lng16 = lng_v[pl.ds(tb, 16)]
                lat16 = lat_v[pl.ds(tb, 16)]
                for j in range(16):
                    lng_s = lng16[j]
                    lat_s = lat16[j]
                    t = tb + j
                    g0 = rows_v[t, 0:16]
                    y0 = jnp.minimum(g0 + lng_s * w00 + lat_s * w10, 80.0)
                    e0 = jnp.exp(y0)
                    rows_v[t, 0:16] = (e0 - 1.0) / (e0 + 1.0)
                    g1 = rows_v[t, 16:32]
                    y1 = jnp.minimum(g1 + lng_s * w01 + lat_s * w11, 80.0)
                    e1 = jnp.exp(y1)
                    rows_v[t, 16:32] = (e1 - 1.0) / (e1 + 1.0)
                return carry

            lax.fori_loop(0, CHUNK // 16, body, 0)
            r0 = row_base + c * CHUNK_ROWS
            for r in range(CHUNK_ROWS):
                pltpu.sync_copy(rows_v.at[pl.ds(r * L, L)], out_hbm.at[r0 + r])
            return chunk_carry

        lax.fori_loop(0, NCHUNK, chunk_body, 0)

    return k(t2, ids3d, lng, lat, wc2)


def kernel(lngs, lats, grid_id, emb_table, W, b):
    t2 = _tc_fold_table(emb_table, W, b.reshape(1, EMB))
    wc2 = W[0:2, :] * 2.0
    ids3d = grid_id.astype(jnp.int32).reshape(N // CHUNK, IDX_ROWS, 128)
    return _sc_gather_affine_tanh(
        t2, ids3d, lngs.reshape(N), lats.reshape(N), wc2)
